# Initial kernel scaffold; baseline (speedup 1.0000x reference)
import jax
import jax.numpy as jnp
from jax.experimental import pallas as pl


def kernel(m1, m2, polar, indices):
    raise NotImplementedError("write your pallas kernel here")


# speedup vs baseline: 2.3919x; 2.3919x over previous
import jax
import jax.numpy as jnp
from jax import lax
from jax.experimental import pallas as pl
from jax.experimental.pallas import tpu as pltpu
from jax.experimental.pallas import tpu_sc as plsc

D = 128
N = 10000
E = 320000
NC = 2
NS = 16
NW = NC * NS
ROWS_PER_W = D // NW
CHUNK = 1280
BLOCKS = CHUNK // 16


def _sc_body(m1_hbm, m2_hbm, polar_hbm, ind1_hbm, ind2_hbm, out_hbm,
             m1_v, m2_v, r_v, idx1_v, idx2_v, out_v):
    wid = lax.axis_index("s") * NC + lax.axis_index("c")
    row0 = wid * ROWS_PER_W

    for r in range(ROWS_PER_W):
        sl = pl.ds(r * N, N)
        pltpu.sync_copy(m1_hbm.at[row0 + r], m1_v.at[sl])
        pltpu.sync_copy(m2_hbm.at[row0 + r], m2_v.at[sl])
        pltpu.sync_copy(polar_hbm.at[row0 + r], r_v.at[sl])

    def conv_body(i, _):
        sl = pl.ds(i * 16, 16)
        r_v[sl] = m1_v[sl] / r_v[sl]
        return 0
    lax.fori_loop(0, (ROWS_PER_W * N) // 16, conv_body, 0)

    offs = [jnp.full((16,), r * N, jnp.int32) for r in range(ROWS_PER_W)]

    def chunk_body(c, _):
        e0 = c * CHUNK
        pltpu.sync_copy(ind1_hbm.at[pl.ds(e0, CHUNK)], idx1_v)
        pltpu.sync_copy(ind2_hbm.at[pl.ds(e0, CHUNK)], idx2_v)

        def block_body(j, _):
            sl = pl.ds(j * 16, 16)
            i1 = idx1_v[sl]
            i2 = idx2_v[sl]
            for r in range(ROWS_PER_W):
                i1r = i1 + offs[r]
                i2r = i2 + offs[r]
                a11 = plsc.load_gather(m1_v, [i1r])
                a12 = plsc.load_gather(m1_v, [i2r])
                a21 = plsc.load_gather(m2_v, [i1r])
                a22 = plsc.load_gather(m2_v, [i2r])
                q1 = plsc.load_gather(r_v, [i1r])
                q2 = plsc.load_gather(r_v, [i2r])
                num = (a11 * a22 + a12 * a21) * 1.5
                out_v[pl.ds(r * CHUNK + j * 16, 16)] = num / (q1 + q2)
            return 0
        lax.fori_loop(0, BLOCKS, block_body, 0)

        for r in range(ROWS_PER_W):
            pltpu.sync_copy(out_v.at[pl.ds(r * CHUNK, CHUNK)],
                            out_hbm.at[row0 + r, pl.ds(e0, CHUNK)])
        return 0
    lax.fori_loop(0, E // CHUNK, chunk_body, 0)


def kernel(m1, m2, polar, indices):
    ind1 = indices[0, :].astype(jnp.int32)
    ind2 = indices[1, :].astype(jnp.int32)
    mesh = plsc.VectorSubcoreMesh(core_axis_name="c", subcore_axis_name="s")
    f = pl.kernel(
        _sc_body,
        out_type=jax.ShapeDtypeStruct((D, E), jnp.float32),
        mesh=mesh,
        compiler_params=pltpu.CompilerParams(needs_layout_passes=False,
                                             use_tc_tiling_on_sc=False),
        scratch_types=[
            pltpu.VMEM((ROWS_PER_W * N,), jnp.float32),
            pltpu.VMEM((ROWS_PER_W * N,), jnp.float32),
            pltpu.VMEM((ROWS_PER_W * N,), jnp.float32),
            pltpu.VMEM((CHUNK,), jnp.int32),
            pltpu.VMEM((CHUNK,), jnp.int32),
            pltpu.VMEM((ROWS_PER_W * CHUNK,), jnp.float32),
        ],
    )
    return f(m1, m2, polar, ind1, ind2)


# speedup vs baseline: 6.5189x; 2.7254x over previous
import jax
import jax.numpy as jnp
from jax import lax
from jax.experimental import pallas as pl
from jax.experimental.pallas import tpu as pltpu
from jax.experimental.pallas import tpu_sc as plsc

D = 128
N = 10000
E = 320000
NC = 2
NS = 16
NW = NC * NS
RW = D // NW
CHUNK = 800
NCHUNKS = E // CHUNK


def _sc_body(m1_hbm, m2_hbm, polar_hbm, ind1_hbm, ind2_hbm, out_hbm, *refs):
    m1_r = refs[0:RW]
    m2_r = refs[RW:2 * RW]
    rr_r = refs[2 * RW:3 * RW]
    idx1_v, idx2_v, out_v, sem_in0, sem_in1, sem_out0, sem_out1 = refs[3 * RW:]
    sem_in = (sem_in0, sem_in1)
    sem_out = (sem_out0, sem_out1)

    wid = lax.axis_index("s") * NC + lax.axis_index("c")
    row0 = wid * RW

    for r in range(RW):
        pltpu.sync_copy(m1_hbm.at[row0 + r], m1_r[r])
        pltpu.sync_copy(m2_hbm.at[row0 + r], m2_r[r])
        pltpu.sync_copy(polar_hbm.at[row0 + r], rr_r[r])

    for r in range(RW):
        @plsc.parallel_loop(0, N, step=16, unroll=4)
        def conv_body(i, r=r):
            sl = pl.ds(i, 16)
            rr_r[r][sl] = m1_r[r][sl] / rr_r[r][sl]

    def in_copies(c, p):
        e0 = c * CHUNK
        bsl = pl.ds(p * CHUNK, CHUNK)
        return (
            pltpu.make_async_copy(ind1_hbm.at[pl.ds(e0, CHUNK)],
                                  idx1_v.at[bsl], sem_in[p]),
            pltpu.make_async_copy(ind2_hbm.at[pl.ds(e0, CHUNK)],
                                  idx2_v.at[bsl], sem_in[p]),
        )

    def out_copies(c, p):
        e0 = c * CHUNK
        return tuple(
            pltpu.make_async_copy(
                out_v.at[pl.ds((p * RW + r) * CHUNK, CHUNK)],
                out_hbm.at[row0 + r, pl.ds(e0, CHUNK)],
                sem_out[p])
            for r in range(RW)
        )

    for cp in in_copies(0, 0):
        cp.start()

    def pair_body(c2, _):
        for p in (0, 1):
            c = c2 * 2 + p
            for cp in in_copies(c, p):
                cp.wait()
            @pl.when(c < NCHUNKS - 1)
            def _():
                for cp in in_copies(c + 1, 1 - p):
                    cp.start()
            @pl.when(c2 >= 1)
            def _():
                for cp in out_copies(c - 2, p):
                    cp.wait()

            @plsc.parallel_loop(0, CHUNK, step=16, unroll=2)
            def blk(e, p=p):
                i1 = idx1_v[pl.ds(p * CHUNK + e, 16)]
                i2 = idx2_v[pl.ds(p * CHUNK + e, 16)]
                for r in range(RW):
                    a11 = plsc.load_gather(m1_r[r], [i1])
                    a12 = plsc.load_gather(m1_r[r], [i2])
                    a21 = plsc.load_gather(m2_r[r], [i1])
                    a22 = plsc.load_gather(m2_r[r], [i2])
                    q1 = plsc.load_gather(rr_r[r], [i1])
                    q2 = plsc.load_gather(rr_r[r], [i2])
                    num = (a11 * a22 + a12 * a21) * 1.5
                    out_v[pl.ds((p * RW + r) * CHUNK + e, 16)] = num / (q1 + q2)

            for cp in out_copies(c, p):
                cp.start()
        return 0

    lax.fori_loop(0, NCHUNKS // 2, pair_body, 0)

    for p in (0, 1):
        for cp in out_copies(NCHUNKS - 2 + p, p):
            cp.wait()


def kernel(m1, m2, polar, indices):
    ind1 = indices[0, :].astype(jnp.int32)
    ind2 = indices[1, :].astype(jnp.int32)
    mesh = plsc.VectorSubcoreMesh(core_axis_name="c", subcore_axis_name="s")
    f = pl.kernel(
        _sc_body,
        out_type=jax.ShapeDtypeStruct((D, E), jnp.float32),
        mesh=mesh,
        compiler_params=pltpu.CompilerParams(needs_layout_passes=False,
                                             use_tc_tiling_on_sc=False),
        scratch_types=(
            [pltpu.VMEM((N,), jnp.float32) for _ in range(3 * RW)]
            + [
                pltpu.VMEM((2 * CHUNK,), jnp.int32),
                pltpu.VMEM((2 * CHUNK,), jnp.int32),
                pltpu.VMEM((2 * RW * CHUNK,), jnp.float32),
                pltpu.SemaphoreType.DMA,
                pltpu.SemaphoreType.DMA,
                pltpu.SemaphoreType.DMA,
                pltpu.SemaphoreType.DMA,
            ]
        ),
    )
    return f(m1, m2, polar, ind1, ind2)


# speedup vs baseline: 8.3314x; 1.2780x over previous
import jax
import jax.numpy as jnp
from jax import lax
from jax.experimental import pallas as pl
from jax.experimental.pallas import tpu as pltpu
from jax.experimental.pallas import tpu_sc as plsc

D = 128
N = 10000
E = 320000
NC = 2
NS = 16
NW = NC * NS
RW = D // NW
CHUNK = 4000
NCHUNKS = E // CHUNK

_MASK_HI = -65536
_HALF = 0x8000


def _sc_body(m1_hbm, m2_hbm, polar_hbm, ind1_hbm, ind2_hbm, out_hbm, *refs):
    t12_r = refs[0:RW]
    rr_r = refs[RW:2 * RW]
    idx1_v, idx2_v, out_v, sem_in0, sem_in1, sem_out0, sem_out1 = refs[2 * RW:]
    sem_in = (sem_in0, sem_in1)
    sem_out = (sem_out0, sem_out1)

    wid = lax.axis_index("s") * NC + lax.axis_index("c")
    row0 = wid * RW

    mask_hi = jnp.full((16,), _MASK_HI, jnp.int32)
    half = jnp.full((16,), _HALF, jnp.int32)

    for r in range(RW):
        pltpu.sync_copy(m1_hbm.at[row0 + r], rr_r[r])
        pltpu.sync_copy(m2_hbm.at[row0 + r], t12_r[r])
        pltpu.sync_copy(polar_hbm.at[row0 + r], out_v.at[pl.ds(0, N)])

        @plsc.parallel_loop(0, N, step=16, unroll=4)
        def prol(i, r=r):
            sl = pl.ds(i, 16)
            m1v = rr_r[r][sl]
            m2v = t12_r[r][sl]
            pv = out_v[sl]
            b1 = (plsc.bitcast(m1v, jnp.int32) + half) & mask_hi
            b2 = lax.shift_right_logical(plsc.bitcast(m2v, jnp.int32) + half,
                                         16)
            t12_r[r][sl] = plsc.bitcast(b1 | b2, jnp.float32)
            rr_r[r][sl] = m1v / pv

    def in_copies(c, p):
        e0 = c * CHUNK
        bsl = pl.ds(p * CHUNK, CHUNK)
        return (
            pltpu.make_async_copy(ind1_hbm.at[pl.ds(e0, CHUNK)],
                                  idx1_v.at[bsl], sem_in[p]),
            pltpu.make_async_copy(ind2_hbm.at[pl.ds(e0, CHUNK)],
                                  idx2_v.at[bsl], sem_in[p]),
        )

    def out_copies(c, p):
        e0 = c * CHUNK
        return tuple(
            pltpu.make_async_copy(
                out_v.at[pl.ds((p * RW + r) * CHUNK, CHUNK)],
                out_hbm.at[row0 + r, pl.ds(e0, CHUNK)],
                sem_out[p])
            for r in range(RW)
        )

    for cp in in_copies(0, 0):
        cp.start()

    def pair_body(c2, _):
        for p in (0, 1):
            c = c2 * 2 + p
            for cp in in_copies(c, p):
                cp.wait()
            @pl.when(c < NCHUNKS - 1)
            def _():
                for cp in in_copies(c + 1, 1 - p):
                    cp.start()
            @pl.when(c2 >= 1)
            def _():
                for cp in out_copies(c - 2, p):
                    cp.wait()

            @plsc.parallel_loop(0, CHUNK, step=16, unroll=2)
            def blk(e, p=p):
                i1 = idx1_v[pl.ds(p * CHUNK + e, 16)]
                i2 = idx2_v[pl.ds(p * CHUNK + e, 16)]
                for r in range(RW):
                    w1 = plsc.bitcast(plsc.load_gather(t12_r[r], [i1]),
                                      jnp.int32)
                    w2 = plsc.bitcast(plsc.load_gather(t12_r[r], [i2]),
                                      jnp.int32)
                    q1 = plsc.load_gather(rr_r[r], [i1])
                    q2 = plsc.load_gather(rr_r[r], [i2])
                    a11 = plsc.bitcast(w1 & mask_hi, jnp.float32)
                    a21 = plsc.bitcast(lax.shift_left(w1, 16), jnp.float32)
                    a12 = plsc.bitcast(w2 & mask_hi, jnp.float32)
                    a22 = plsc.bitcast(lax.shift_left(w2, 16), jnp.float32)
                    num = (a11 * a22 + a12 * a21) * 1.5
                    out_v[pl.ds((p * RW + r) * CHUNK + e, 16)] = num / (q1 + q2)

            for cp in out_copies(c, p):
                cp.start()
        return 0

    lax.fori_loop(0, NCHUNKS // 2, pair_body, 0)

    for p in (0, 1):
        for cp in out_copies(NCHUNKS - 2 + p, p):
            cp.wait()


def kernel(m1, m2, polar, indices):
    ind1 = indices[0, :].astype(jnp.int32)
    ind2 = indices[1, :].astype(jnp.int32)
    mesh = plsc.VectorSubcoreMesh(core_axis_name="c", subcore_axis_name="s")
    f = pl.kernel(
        _sc_body,
        out_type=jax.ShapeDtypeStruct((D, E), jnp.float32),
        mesh=mesh,
        compiler_params=pltpu.CompilerParams(needs_layout_passes=False,
                                             use_tc_tiling_on_sc=False),
        scratch_types=(
            [pltpu.VMEM((N,), jnp.float32) for _ in range(2 * RW)]
            + [
                pltpu.VMEM((2 * CHUNK,), jnp.int32),
                pltpu.VMEM((2 * CHUNK,), jnp.int32),
                pltpu.VMEM((2 * RW * CHUNK,), jnp.float32),
                pltpu.SemaphoreType.DMA,
                pltpu.SemaphoreType.DMA,
                pltpu.SemaphoreType.DMA,
                pltpu.SemaphoreType.DMA,
            ]
        ),
    )
    return f(m1, m2, polar, ind1, ind2)


# speedup vs baseline: 8.3340x; 1.0003x over previous
import jax
import jax.numpy as jnp
from jax import lax
from jax.experimental import pallas as pl
from jax.experimental.pallas import tpu as pltpu
from jax.experimental.pallas import tpu_sc as plsc

D = 128
N = 10000
E = 320000
NC = 2
NS = 16
NW = NC * NS
RW = D // NW
CHUNK = 4000
NCHUNKS = E // CHUNK

_MASK_HI = -65536
_HALF = 0x8000


def _sc_body(m1_hbm, m2_hbm, polar_hbm, ind1_hbm, ind2_hbm, out_hbm, *refs):
    t12_r = refs[0:RW]
    rr_r = refs[RW:2 * RW]
    idx1_v, idx2_v, out_v, sem_in0, sem_in1, sem_out0, sem_out1 = refs[2 * RW:]
    sem_in = (sem_in0, sem_in1)
    sem_out = (sem_out0, sem_out1)

    wid = lax.axis_index("s") * NC + lax.axis_index("c")
    row0 = wid * RW

    mask_hi = jnp.full((16,), _MASK_HI, jnp.int32)
    half = jnp.full((16,), _HALF, jnp.int32)

    for r in range(RW):
        pltpu.sync_copy(m1_hbm.at[row0 + r], rr_r[r])
        pltpu.sync_copy(m2_hbm.at[row0 + r], t12_r[r])
        pltpu.sync_copy(polar_hbm.at[row0 + r], out_v.at[pl.ds(0, N)])

        @plsc.parallel_loop(0, N, step=16, unroll=4)
        def prol(i, r=r):
            sl = pl.ds(i, 16)
            m1v = rr_r[r][sl]
            m2v = t12_r[r][sl]
            pv = out_v[sl]
            b1 = (plsc.bitcast(m1v, jnp.int32) + half) & mask_hi
            b2 = lax.shift_right_logical(plsc.bitcast(m2v, jnp.int32) + half,
                                         16)
            t12_r[r][sl] = plsc.bitcast(b1 | b2, jnp.float32)
            rr_r[r][sl] = m1v / pv

    def in_copies(c, p):
        e0 = c * CHUNK
        bsl = pl.ds(p * CHUNK, CHUNK)
        return (
            pltpu.make_async_copy(ind1_hbm.at[pl.ds(e0, CHUNK)],
                                  idx1_v.at[bsl], sem_in[p]),
            pltpu.make_async_copy(ind2_hbm.at[pl.ds(e0, CHUNK)],
                                  idx2_v.at[bsl], sem_in[p]),
        )

    def out_copies(c, p):
        e0 = c * CHUNK
        return tuple(
            pltpu.make_async_copy(
                out_v.at[pl.ds((p * RW + r) * CHUNK, CHUNK)],
                out_hbm.at[row0 + r, pl.ds(e0, CHUNK)],
                sem_out[p])
            for r in range(RW)
        )

    for cp in in_copies(0, 0):
        cp.start()

    def pair_body(c2, _):
        for p in (0, 1):
            c = c2 * 2 + p
            for cp in in_copies(c, p):
                cp.wait()
            @pl.when(c < NCHUNKS - 1)
            def _():
                for cp in in_copies(c + 1, 1 - p):
                    cp.start()
            @pl.when(c2 >= 1)
            def _():
                for cp in out_copies(c - 2, p):
                    cp.wait()

            @plsc.parallel_loop(0, CHUNK, step=16, unroll=4)
            def blk(e, p=p):
                i1 = idx1_v[pl.ds(p * CHUNK + e, 16)]
                i2 = idx2_v[pl.ds(p * CHUNK + e, 16)]
                for r in range(RW):
                    w1 = plsc.bitcast(plsc.load_gather(t12_r[r], [i1]),
                                      jnp.int32)
                    w2 = plsc.bitcast(plsc.load_gather(t12_r[r], [i2]),
                                      jnp.int32)
                    q1 = plsc.load_gather(rr_r[r], [i1])
                    q2 = plsc.load_gather(rr_r[r], [i2])
                    a11 = plsc.bitcast(w1 & mask_hi, jnp.float32)
                    a21 = plsc.bitcast(lax.shift_left(w1, 16), jnp.float32)
                    a12 = plsc.bitcast(w2 & mask_hi, jnp.float32)
                    a22 = plsc.bitcast(lax.shift_left(w2, 16), jnp.float32)
                    num = (a11 * a22 + a12 * a21) * 1.5
                    out_v[pl.ds((p * RW + r) * CHUNK + e, 16)] = num / (q1 + q2)

            for cp in out_copies(c, p):
                cp.start()
        return 0

    lax.fori_loop(0, NCHUNKS // 2, pair_body, 0)

    for p in (0, 1):
        for cp in out_copies(NCHUNKS - 2 + p, p):
            cp.wait()


def kernel(m1, m2, polar, indices):
    ind1 = indices[0, :].astype(jnp.int32)
    ind2 = indices[1, :].astype(jnp.int32)
    mesh = plsc.VectorSubcoreMesh(core_axis_name="c", subcore_axis_name="s")
    f = pl.kernel(
        _sc_body,
        out_type=jax.ShapeDtypeStruct((D, E), jnp.float32),
        mesh=mesh,
        compiler_params=pltpu.CompilerParams(needs_layout_passes=False,
                                             use_tc_tiling_on_sc=False),
        scratch_types=(
            [pltpu.VMEM((N,), jnp.float32) for _ in range(2 * RW)]
            + [
                pltpu.VMEM((2 * CHUNK,), jnp.int32),
                pltpu.VMEM((2 * CHUNK,), jnp.int32),
                pltpu.VMEM((2 * RW * CHUNK,), jnp.float32),
                pltpu.SemaphoreType.DMA,
                pltpu.SemaphoreType.DMA,
                pltpu.SemaphoreType.DMA,
                pltpu.SemaphoreType.DMA,
            ]
        ),
    )
    return f(m1, m2, polar, ind1, ind2)


# speedup vs baseline: 8.6977x; 1.0436x over previous
import jax
import jax.numpy as jnp
from jax import lax
from jax.experimental import pallas as pl
from jax.experimental.pallas import tpu as pltpu
from jax.experimental.pallas import tpu_sc as plsc

D = 128
N = 10000
E = 320000
NC = 2
NS = 16
NW = NC * NS
RW = D // NW
NP = RW // 2
CHUNK = 4000
NCHUNKS = E // CHUNK

_MASK_HI = -65536
_HALF = 0x8000
_FMAX = 3.0e38


def _sc_body(m1_hbm, m2_hbm, polar_hbm, ind1_hbm, ind2_hbm, out_hbm,
             tA0, tA1, tB0, tB1, tC0, tC1, idx1_v, idx2_v, out_v,
             sem_in0, sem_in1, sem_out0, sem_out1):
    tA = (tA0, tA1)
    tB = (tB0, tB1)
    tC = (tC0, tC1)
    sem_in = (sem_in0, sem_in1)
    sem_out = (sem_out0, sem_out1)

    wid = lax.axis_index("s") * NC + lax.axis_index("c")
    row0 = wid * RW

    mask_hi = jnp.full((16,), _MASK_HI, jnp.int32)
    half = jnp.full((16,), _HALF, jnp.int32)
    fmax = jnp.full((16,), _FMAX, jnp.float32)

    def pack2(hi, lo):
        b1 = (plsc.bitcast(jnp.minimum(hi, fmax), jnp.int32) + half) & mask_hi
        b2 = lax.shift_right_logical(
            plsc.bitcast(jnp.minimum(lo, fmax), jnp.int32) + half, 16)
        return plsc.bitcast(b1 | b2, jnp.float32)

    for g in range(NP):
        r0 = row0 + 2 * g
        pltpu.sync_copy(m1_hbm.at[r0], tA[g])
        pltpu.sync_copy(m1_hbm.at[r0 + 1], tB[g])
        pltpu.sync_copy(polar_hbm.at[r0], tC[g])
        pltpu.sync_copy(polar_hbm.at[r0 + 1], out_v.at[pl.ds(0, N)])
        pltpu.sync_copy(m2_hbm.at[r0], out_v.at[pl.ds(N, N)])
        pltpu.sync_copy(m2_hbm.at[r0 + 1], out_v.at[pl.ds(2 * N, N)])

        @plsc.parallel_loop(0, N, step=16, unroll=4)
        def prol(i, g=g):
            sl = pl.ds(i, 16)
            m1a = tA[g][sl]
            m1b = tB[g][sl]
            pa = tC[g][sl]
            pb = out_v[sl]
            m2a = out_v[pl.ds(N + i, 16)]
            m2b = out_v[pl.ds(2 * N + i, 16)]
            tA[g][sl] = pack2(m1a, m1b)
            tB[g][sl] = pack2(m2a, m2b)
            tC[g][sl] = pack2(m1a / pa, m1b / pb)

    def in_copies(c, p):
        e0 = c * CHUNK
        bsl = pl.ds(p * CHUNK, CHUNK)
        return (
            pltpu.make_async_copy(ind1_hbm.at[pl.ds(e0, CHUNK)],
                                  idx1_v.at[bsl], sem_in[p]),
            pltpu.make_async_copy(ind2_hbm.at[pl.ds(e0, CHUNK)],
                                  idx2_v.at[bsl], sem_in[p]),
        )

    def out_copies(c, p):
        e0 = c * CHUNK
        return tuple(
            pltpu.make_async_copy(
                out_v.at[pl.ds((p * RW + r) * CHUNK, CHUNK)],
                out_hbm.at[row0 + r, pl.ds(e0, CHUNK)],
                sem_out[p])
            for r in range(RW)
        )

    for cp in in_copies(0, 0):
        cp.start()

    def pair_body(c2, _):
        for p in (0, 1):
            c = c2 * 2 + p
            for cp in in_copies(c, p):
                cp.wait()
            @pl.when(c < NCHUNKS - 1)
            def _():
                for cp in in_copies(c + 1, 1 - p):
                    cp.start()
            @pl.when(c2 >= 1)
            def _():
                for cp in out_copies(c - 2, p):
                    cp.wait()

            @plsc.parallel_loop(0, CHUNK, step=16, unroll=2)
            def blk(e, p=p):
                i1 = idx1_v[pl.ds(p * CHUNK + e, 16)]
                i2 = idx2_v[pl.ds(p * CHUNK + e, 16)]
                for g in range(NP):
                    wA1 = plsc.bitcast(plsc.load_gather(tA[g], [i1]),
                                       jnp.int32)
                    wA2 = plsc.bitcast(plsc.load_gather(tA[g], [i2]),
                                       jnp.int32)
                    wB1 = plsc.bitcast(plsc.load_gather(tB[g], [i1]),
                                       jnp.int32)
                    wB2 = plsc.bitcast(plsc.load_gather(tB[g], [i2]),
                                       jnp.int32)
                    wC1 = plsc.bitcast(plsc.load_gather(tC[g], [i1]),
                                       jnp.int32)
                    wC2 = plsc.bitcast(plsc.load_gather(tC[g], [i2]),
                                       jnp.int32)
                    for h in range(2):
                        if h == 0:
                            uA1 = plsc.bitcast(wA1 & mask_hi, jnp.float32)
                            uA2 = plsc.bitcast(wA2 & mask_hi, jnp.float32)
                            uB1 = plsc.bitcast(wB1 & mask_hi, jnp.float32)
                            uB2 = plsc.bitcast(wB2 & mask_hi, jnp.float32)
                            uC1 = plsc.bitcast(wC1 & mask_hi, jnp.float32)
                            uC2 = plsc.bitcast(wC2 & mask_hi, jnp.float32)
                        else:
                            uA1 = plsc.bitcast(lax.shift_left(wA1, 16),
                                               jnp.float32)
                            uA2 = plsc.bitcast(lax.shift_left(wA2, 16),
                                               jnp.float32)
                            uB1 = plsc.bitcast(lax.shift_left(wB1, 16),
                                               jnp.float32)
                            uB2 = plsc.bitcast(lax.shift_left(wB2, 16),
                                               jnp.float32)
                            uC1 = plsc.bitcast(lax.shift_left(wC1, 16),
                                               jnp.float32)
                            uC2 = plsc.bitcast(lax.shift_left(wC2, 16),
                                               jnp.float32)
                        num = (uA1 * uB2 + uA2 * uB1) * 1.5
                        r = 2 * g + h
                        out_v[pl.ds((p * RW + r) * CHUNK + e, 16)] = (
                            num / (uC1 + uC2))

            for cp in out_copies(c, p):
                cp.start()
        return 0

    lax.fori_loop(0, NCHUNKS // 2, pair_body, 0)

    for p in (0, 1):
        for cp in out_copies(NCHUNKS - 2 + p, p):
            cp.wait()


def kernel(m1, m2, polar, indices):
    ind1 = indices[0, :].astype(jnp.int32)
    ind2 = indices[1, :].astype(jnp.int32)
    mesh = plsc.VectorSubcoreMesh(core_axis_name="c", subcore_axis_name="s")
    f = pl.kernel(
        _sc_body,
        out_type=jax.ShapeDtypeStruct((D, E), jnp.float32),
        mesh=mesh,
        compiler_params=pltpu.CompilerParams(needs_layout_passes=False,
                                             use_tc_tiling_on_sc=False),
        scratch_types=(
            [pltpu.VMEM((N,), jnp.float32) for _ in range(3 * NP)]
            + [
                pltpu.VMEM((2 * CHUNK,), jnp.int32),
                pltpu.VMEM((2 * CHUNK,), jnp.int32),
                pltpu.VMEM((2 * RW * CHUNK,), jnp.float32),
                pltpu.SemaphoreType.DMA,
                pltpu.SemaphoreType.DMA,
                pltpu.SemaphoreType.DMA,
                pltpu.SemaphoreType.DMA,
            ]
        ),
    )
    return f(m1, m2, polar, ind1, ind2)


# speedup vs baseline: 9.3789x; 1.0783x over previous
import jax
import jax.numpy as jnp
from jax import lax
from jax.experimental import pallas as pl
from jax.experimental.pallas import tpu as pltpu
from jax.experimental.pallas import tpu_sc as plsc

D = 128
N = 10000
E = 320000
NC = 2
NS = 16
NW = NC * NS
RW = D // NW
NP = RW // 2
CHUNK = 4000
NCHUNKS = E // CHUNK

_MASK_HI = -65536
_HALF = 0x8000
_FMAX = 3.0e38


def _sc_body(m1_hbm, m2_hbm, polar_hbm, ind1_hbm, ind2_hbm, out_hbm,
             tA0, tA1, tB0, tB1, tC0, tC1, idx1_v, idx2_v, out_v,
             sem_in0, sem_in1, sem_out0, sem_out1):
    tA = (tA0, tA1)
    tB = (tB0, tB1)
    tC = (tC0, tC1)
    sem_in = (sem_in0, sem_in1)
    sem_out = (sem_out0, sem_out1)

    wid = lax.axis_index("s") * NC + lax.axis_index("c")
    row0 = wid * RW

    def pack2(a, b):
        return plsc.bitcast(
            plsc.pack(a, b, format=plsc.PackFormat.INTERLEAVED), jnp.float32)

    for g in range(NP):
        r0 = row0 + 2 * g
        pltpu.sync_copy(m1_hbm.at[r0], tA[g])
        pltpu.sync_copy(m1_hbm.at[r0 + 1], tB[g])
        pltpu.sync_copy(polar_hbm.at[r0], tC[g])
        pltpu.sync_copy(polar_hbm.at[r0 + 1], out_v.at[pl.ds(0, N)])
        pltpu.sync_copy(m2_hbm.at[r0], out_v.at[pl.ds(N, N)])
        pltpu.sync_copy(m2_hbm.at[r0 + 1], out_v.at[pl.ds(2 * N, N)])

        @plsc.parallel_loop(0, N, step=16, unroll=4)
        def prol(i, g=g):
            sl = pl.ds(i, 16)
            m1a = tA[g][sl]
            m1b = tB[g][sl]
            pa = tC[g][sl]
            pb = out_v[sl]
            m2a = out_v[pl.ds(N + i, 16)]
            m2b = out_v[pl.ds(2 * N + i, 16)]
            tA[g][sl] = pack2(m1a, m1b)
            tB[g][sl] = pack2(m2a, m2b)
            tC[g][sl] = pack2(m1a / (1.5 * pa), m1b / (1.5 * pb))

    def in_copies(c, p):
        e0 = c * CHUNK
        bsl = pl.ds(p * CHUNK, CHUNK)
        return (
            pltpu.make_async_copy(ind1_hbm.at[pl.ds(e0, CHUNK)],
                                  idx1_v.at[bsl], sem_in[p]),
            pltpu.make_async_copy(ind2_hbm.at[pl.ds(e0, CHUNK)],
                                  idx2_v.at[bsl], sem_in[p]),
        )

    def out_copies(c, p):
        e0 = c * CHUNK
        return tuple(
            pltpu.make_async_copy(
                out_v.at[pl.ds((p * RW + r) * CHUNK, CHUNK)],
                out_hbm.at[row0 + r, pl.ds(e0, CHUNK)],
                sem_out[p])
            for r in range(RW)
        )

    for cp in in_copies(0, 0):
        cp.start()

    def pair_body(c2, _):
        for p in (0, 1):
            c = c2 * 2 + p
            for cp in in_copies(c, p):
                cp.wait()
            @pl.when(c < NCHUNKS - 1)
            def _():
                for cp in in_copies(c + 1, 1 - p):
                    cp.start()
            @pl.when(c2 >= 1)
            def _():
                for cp in out_copies(c - 2, p):
                    cp.wait()

            @plsc.parallel_loop(0, CHUNK, step=16, unroll=2)
            def blk(e, p=p):
                i1 = idx1_v[pl.ds(p * CHUNK + e, 16)]
                i2 = idx2_v[pl.ds(p * CHUNK + e, 16)]
                for g in range(NP):
                    bf = jnp.bfloat16
                    xA1 = plsc.bitcast(plsc.load_gather(tA[g], [i1]), bf)
                    xA2 = plsc.bitcast(plsc.load_gather(tA[g], [i2]), bf)
                    xB1 = plsc.bitcast(plsc.load_gather(tB[g], [i1]), bf)
                    xB2 = plsc.bitcast(plsc.load_gather(tB[g], [i2]), bf)
                    xC1 = plsc.bitcast(plsc.load_gather(tC[g], [i1]), bf)
                    xC2 = plsc.bitcast(plsc.load_gather(tC[g], [i2]), bf)
                    nm = xA1 * xB2 + xA2 * xB1
                    dn = xC1 + xC2
                    n0, n1 = plsc.unpack(nm, format=plsc.PackFormat.INTERLEAVED)
                    d0, d1 = plsc.unpack(dn, format=plsc.PackFormat.INTERLEAVED)
                    r = 2 * g
                    out_v[pl.ds((p * RW + r) * CHUNK + e, 16)] = n0 / d0
                    out_v[pl.ds((p * RW + r + 1) * CHUNK + e, 16)] = n1 / d1

            for cp in out_copies(c, p):
                cp.start()
        return 0

    lax.fori_loop(0, NCHUNKS // 2, pair_body, 0)

    for p in (0, 1):
        for cp in out_copies(NCHUNKS - 2 + p, p):
            cp.wait()


def kernel(m1, m2, polar, indices):
    ind1 = indices[0, :].astype(jnp.int32)
    ind2 = indices[1, :].astype(jnp.int32)
    mesh = plsc.VectorSubcoreMesh(core_axis_name="c", subcore_axis_name="s")
    f = pl.kernel(
        _sc_body,
        out_type=jax.ShapeDtypeStruct((D, E), jnp.float32),
        mesh=mesh,
        compiler_params=pltpu.CompilerParams(needs_layout_passes=False,
                                             use_tc_tiling_on_sc=False),
        scratch_types=(
            [pltpu.VMEM((N,), jnp.float32) for _ in range(3 * NP)]
            + [
                pltpu.VMEM((2 * CHUNK,), jnp.int32),
                pltpu.VMEM((2 * CHUNK,), jnp.int32),
                pltpu.VMEM((2 * RW * CHUNK,), jnp.float32),
                pltpu.SemaphoreType.DMA,
                pltpu.SemaphoreType.DMA,
                pltpu.SemaphoreType.DMA,
                pltpu.SemaphoreType.DMA,
            ]
        ),
    )
    return f(m1, m2, polar, ind1, ind2)
